# baseline (speedup 1.0000x reference)
import functools

import jax
import jax.numpy as jnp
import numpy as np
from jax import lax
from jax.experimental import pallas as pl
from jax.experimental.pallas import tpu as pltpu
from jax.experimental.pallas import tpu_sc as plsc

VOCAB = 1000000
D = 64
SEQ = 2048
BATCH = 4
N = SEQ * BATCH

NC = 2
NS = 16
NW = NC * NS
CHUNK = 128
CPW = N // (NW * CHUNK)
ROWS_PW = N // NW


def _pos_enc_np(max_len, d_model):
    position = np.arange(max_len, dtype=np.float32)[:, None]
    div_term = np.exp(
        np.arange(0, d_model, 2, dtype=np.float32) * -(np.log(10000.0) / d_model))
    pe = np.zeros((max_len, d_model), dtype=np.float32)
    pe[:, 0::2] = np.sin(position * div_term)
    pe[:, 1::2] = np.cos(position * div_term)
    return pe


_PE_FULL = np.repeat(_pos_enc_np(SEQ, D), BATCH, axis=0)


@functools.cache
def _make_gather():
    mesh = plsc.VectorSubcoreMesh(core_axis_name="c", subcore_axis_name="s")

    @functools.partial(
        pl.kernel,
        mesh=mesh,
        out_type=jax.ShapeDtypeStruct((N, D), jnp.float32),
        scratch_types=[
            pltpu.VMEM((CPW, CHUNK), jnp.int32),
            pltpu.VMEM((CPW, CHUNK, D), jnp.float32),
            pltpu.SemaphoreType.DMA,
        ],
        compiler_params=pltpu.CompilerParams(use_tc_tiling_on_sc=False),
    )
    def gather_k(table_hbm, idx_hbm, out_hbm, idx_v, rows_v, sem):
        wid = lax.axis_index("s") * NC + lax.axis_index("c")
        pltpu.sync_copy(idx_hbm.at[pl.ds(wid * CPW, CPW)], idx_v)
        copies = []
        for c in range(CPW):
            copies.append(
                pltpu.async_copy(table_hbm.at[idx_v.at[c]], rows_v.at[c], sem))
        for c in range(CPW):
            copies[c].wait()
        for c in range(CPW):
            pltpu.sync_copy(
                rows_v.at[c], out_hbm.at[pl.ds((wid * CPW + c) * CHUNK, CHUNK)])

    return gather_k


def _ln_body(emb_ref, pe_ref, scale_ref, bias_ref, out_ref):
    e = emb_ref[...] + pe_ref[...]
    mu = jnp.mean(e, axis=1, keepdims=True)
    d = e - mu
    var = jnp.mean(d * d, axis=1, keepdims=True)
    inv = lax.rsqrt(var + 1e-10)
    out_ref[...] = d * inv * scale_ref[...] + bias_ref[...]


def _ln(emb, pe_full, scale, bias):
    blk = 1024
    grid = (N // blk,)
    return pl.pallas_call(
        _ln_body,
        grid=grid,
        in_specs=[
            pl.BlockSpec((blk, D), lambda i: (i, 0)),
            pl.BlockSpec((blk, D), lambda i: (i, 0)),
            pl.BlockSpec((1, D), lambda i: (0, 0)),
            pl.BlockSpec((1, D), lambda i: (0, 0)),
        ],
        out_specs=pl.BlockSpec((blk, D), lambda i: (i, 0)),
        out_shape=jax.ShapeDtypeStruct((N, D), jnp.float32),
    )(emb, pe_full, scale, bias)


def kernel(x, table, ln_scale, ln_bias):
    idx = x.reshape(-1).astype(jnp.int32).reshape(NW * CPW, CHUNK)
    emb = _make_gather()(table, idx)
    pe_full = jnp.asarray(_PE_FULL)
    out = _ln(emb, pe_full, ln_scale.reshape(1, D), ln_bias.reshape(1, D))
    return out.reshape(SEQ, BATCH, D)


# speedup vs baseline: 1.2795x; 1.2795x over previous
import functools

import jax
import jax.numpy as jnp
import numpy as np
from jax import lax
from jax.experimental import pallas as pl
from jax.experimental.pallas import tpu as pltpu
from jax.experimental.pallas import tpu_sc as plsc

VOCAB = 1000000
D = 64
SEQ = 2048
BATCH = 4
N = SEQ * BATCH

NC = 2
NS = 16
NW = NC * NS
CHUNK = 128
CPW = N // (NW * CHUNK)
ROWS_PW = N // NW


def _pos_enc_np(max_len, d_model):
    position = np.arange(max_len, dtype=np.float32)[:, None]
    div_term = np.exp(
        np.arange(0, d_model, 2, dtype=np.float32) * -(np.log(10000.0) / d_model))
    pe = np.zeros((max_len, d_model), dtype=np.float32)
    pe[:, 0::2] = np.sin(position * div_term)
    pe[:, 1::2] = np.cos(position * div_term)
    return pe


_PE_FULL = np.repeat(_pos_enc_np(SEQ, D), BATCH, axis=0)


DMA_CHUNK = 16


@functools.cache
def _make_gather():
    mesh = plsc.VectorSubcoreMesh(core_axis_name="c", subcore_axis_name="s")

    @functools.partial(
        pl.kernel,
        mesh=mesh,
        out_type=jax.ShapeDtypeStruct((N, D), jnp.float32),
        scratch_types=[
            pltpu.VMEM((ROWS_PW,), jnp.int32),
            pltpu.SemaphoreType.DMA,
        ],
    )
    def gather_k(table_hbm, idx_hbm, out_hbm, idx_v, sem):
        wid = lax.axis_index("s") * NC + lax.axis_index("c")
        base = wid * ROWS_PW
        pltpu.sync_copy(idx_hbm.at[pl.ds(base, ROWS_PW)], idx_v)

        def body(c, carry):
            cbase = c * DMA_CHUNK
            iv = idx_v[pl.ds(cbase, DMA_CHUNK)]
            copies = []
            for j in range(DMA_CHUNK):
                r = iv[j]
                copies.append(pltpu.async_copy(
                    table_hbm.at[pl.ds(r, 1)],
                    out_hbm.at[pl.ds(base + cbase + j, 1)],
                    sem))
            for cp in copies:
                cp.wait()
            return carry

        lax.fori_loop(0, ROWS_PW // DMA_CHUNK, body, 0)

    return gather_k


def _ln_body(emb_ref, pe_ref, scale_ref, bias_ref, out_ref):
    e = emb_ref[...] + pe_ref[...]
    mu = jnp.mean(e, axis=1, keepdims=True)
    d = e - mu
    var = jnp.mean(d * d, axis=1, keepdims=True)
    inv = lax.rsqrt(var + 1e-10)
    out_ref[...] = d * inv * scale_ref[...] + bias_ref[...]


def _ln(emb, pe_full, scale, bias):
    blk = 1024
    grid = (N // blk,)
    return pl.pallas_call(
        _ln_body,
        grid=grid,
        in_specs=[
            pl.BlockSpec((blk, D), lambda i: (i, 0)),
            pl.BlockSpec((blk, D), lambda i: (i, 0)),
            pl.BlockSpec((1, D), lambda i: (0, 0)),
            pl.BlockSpec((1, D), lambda i: (0, 0)),
        ],
        out_specs=pl.BlockSpec((blk, D), lambda i: (i, 0)),
        out_shape=jax.ShapeDtypeStruct((N, D), jnp.float32),
    )(emb, pe_full, scale, bias)


def kernel(x, table, ln_scale, ln_bias):
    idx = x.reshape(-1).astype(jnp.int32)
    emb = _make_gather()(table, idx)
    pe_full = jnp.asarray(_PE_FULL)
    out = _ln(emb, pe_full, ln_scale.reshape(1, D), ln_bias.reshape(1, D))
    return out.reshape(SEQ, BATCH, D)


# speedup vs baseline: 1.7098x; 1.3363x over previous
import functools

import jax
import jax.numpy as jnp
import numpy as np
from jax import lax
from jax.experimental import pallas as pl
from jax.experimental.pallas import tpu as pltpu
from jax.experimental.pallas import tpu_sc as plsc

VOCAB = 1000000
D = 64
SEQ = 2048
BATCH = 4
N = SEQ * BATCH

NC = 2
NS = 16
NW = NC * NS
CHUNK = 128
CPW = N // (NW * CHUNK)
ROWS_PW = N // NW


def _pos_enc_np(max_len, d_model):
    position = np.arange(max_len, dtype=np.float32)[:, None]
    div_term = np.exp(
        np.arange(0, d_model, 2, dtype=np.float32) * -(np.log(10000.0) / d_model))
    pe = np.zeros((max_len, d_model), dtype=np.float32)
    pe[:, 0::2] = np.sin(position * div_term)
    pe[:, 1::2] = np.cos(position * div_term)
    return pe


_PE_FULL = np.repeat(_pos_enc_np(SEQ, D), BATCH, axis=0)


DMA_CHUNK = 16


@functools.cache
def _make_gather():
    mesh = plsc.VectorSubcoreMesh(core_axis_name="c", subcore_axis_name="s")

    @functools.partial(
        pl.kernel,
        mesh=mesh,
        out_type=jax.ShapeDtypeStruct((N, D), jnp.float32),
        scratch_types=[
            pltpu.VMEM((ROWS_PW,), jnp.int32),
            pltpu.VMEM((ROWS_PW, D), jnp.float32),
            pltpu.SemaphoreType.DMA,
        ],
    )
    def gather_k(table_hbm, idx_hbm, out_hbm, idx_v, rows_v, sem):
        wid = lax.axis_index("s") * NC + lax.axis_index("c")
        base = wid * ROWS_PW
        pltpu.sync_copy(idx_hbm.at[pl.ds(base, ROWS_PW)], idx_v)

        def fire(c, carry):
            cbase = c * DMA_CHUNK
            iv = idx_v[pl.ds(cbase, DMA_CHUNK)]
            for j in range(DMA_CHUNK):
                pltpu.async_copy(
                    table_hbm.at[pl.ds(iv[j], 1)],
                    rows_v.at[pl.ds(cbase + j, 1)],
                    sem)
            return carry

        lax.fori_loop(0, ROWS_PW // DMA_CHUNK, fire, 0)

        def drain(c, carry):
            cbase = c * DMA_CHUNK
            for j in range(DMA_CHUNK):
                pltpu.make_async_copy(
                    table_hbm.at[pl.ds(0, 1)],
                    rows_v.at[pl.ds(cbase + j, 1)],
                    sem).wait()
            return carry

        lax.fori_loop(0, ROWS_PW // DMA_CHUNK, drain, 0)
        pltpu.sync_copy(rows_v, out_hbm.at[pl.ds(base, ROWS_PW)])

    return gather_k


def _ln_body(emb_ref, pe_ref, scale_ref, bias_ref, out_ref):
    e = emb_ref[...] + pe_ref[...]
    mu = jnp.mean(e, axis=1, keepdims=True)
    d = e - mu
    var = jnp.mean(d * d, axis=1, keepdims=True)
    inv = lax.rsqrt(var + 1e-10)
    out_ref[...] = d * inv * scale_ref[...] + bias_ref[...]


def _ln(emb, pe_full, scale, bias):
    blk = 1024
    grid = (N // blk,)
    return pl.pallas_call(
        _ln_body,
        grid=grid,
        in_specs=[
            pl.BlockSpec((blk, D), lambda i: (i, 0)),
            pl.BlockSpec((blk, D), lambda i: (i, 0)),
            pl.BlockSpec((1, D), lambda i: (0, 0)),
            pl.BlockSpec((1, D), lambda i: (0, 0)),
        ],
        out_specs=pl.BlockSpec((blk, D), lambda i: (i, 0)),
        out_shape=jax.ShapeDtypeStruct((N, D), jnp.float32),
    )(emb, pe_full, scale, bias)


def kernel(x, table, ln_scale, ln_bias):
    idx = x.reshape(-1).astype(jnp.int32)
    emb = _make_gather()(table, idx)
    pe_full = jnp.asarray(_PE_FULL)
    out = _ln(emb, pe_full, ln_scale.reshape(1, D), ln_bias.reshape(1, D))
    return out.reshape(SEQ, BATCH, D)


# speedup vs baseline: 5.0315x; 2.9428x over previous
import functools

import jax
import jax.numpy as jnp
import numpy as np
from jax import lax
from jax.experimental import pallas as pl
from jax.experimental.pallas import tpu as pltpu
from jax.experimental.pallas import tpu_sc as plsc

VOCAB = 1000000
D = 64
SEQ = 2048
BATCH = 4
N = SEQ * BATCH

NC = 2
NS = 16
NW = NC * NS
TPW = N // NW
LANES = 128
RING = 6
IPAD = TPW + 16


def _pos_enc_np(max_len, d_model):
    position = np.arange(max_len, dtype=np.float32)[:, None]
    div_term = np.exp(
        np.arange(0, d_model, 2, dtype=np.float32) * -(np.log(10000.0) / d_model))
    pe = np.zeros((max_len, d_model), dtype=np.float32)
    pe[:, 0::2] = np.sin(position * div_term)
    pe[:, 1::2] = np.cos(position * div_term)
    return pe


_PE_FULL = np.repeat(_pos_enc_np(SEQ, D), BATCH, axis=0)


@functools.cache
def _make_gather():
    mesh = plsc.VectorSubcoreMesh(core_axis_name="c", subcore_axis_name="s")

    @functools.partial(
        pl.kernel,
        mesh=mesh,
        out_type=jax.ShapeDtypeStruct((N * D,), jnp.float32),
        scratch_types=[
            pltpu.VMEM((IPAD,), jnp.int32),
            pltpu.VMEM((IPAD,), jnp.int32),
            pltpu.VMEM((RING, D, LANES), jnp.float32),
            pltpu.VMEM((TPW * D,), jnp.float32),
            pltpu.SemaphoreType.DMA,
        ],
        compiler_params=pltpu.CompilerParams(needs_layout_passes=False),
    )
    def gather_k(tab_t, q_hbm, m_hbm, out1d, q_v, m_v, ring, rows, sem):
        wid = lax.axis_index("s") * NC + lax.axis_index("c")
        base = wid * TPW
        pltpu.sync_copy(q_hbm.at[pl.ds(base, TPW)], q_v.at[pl.ds(0, TPW)])
        pltpu.sync_copy(m_hbm.at[pl.ds(base, TPW)], m_v.at[pl.ds(0, TPW)])

        def fetch(t, slot):
            qv = q_v[pl.ds(t, 16)][0]
            off = pl.multiple_of(qv * LANES, LANES)
            pltpu.async_copy(
                tab_t.at[:, pl.ds(off, LANES)], ring.at[slot], sem)

        for r in range(RING):
            fetch(r, r)

        lanes16 = lax.iota(jnp.int32, 16)

        def body(t, carry):
            slot = lax.rem(t, RING)
            pltpu.make_async_copy(
                tab_t.at[:, pl.ds(0, LANES)], ring.at[slot], sem).wait()
            m = m_v[pl.ds(t, 16)][0]
            midx = jnp.full((16,), m, jnp.int32)
            sidx = jnp.full((16,), slot, jnp.int32)
            for k in range(D // 16):
                g = plsc.load_gather(ring, [sidx, lanes16 + 16 * k, midx])
                rows[pl.ds(t * D + 16 * k, 16)] = g
            tn = t + RING

            @pl.when(tn < TPW)
            def _():
                fetch(tn, lax.rem(tn, RING))

            return carry

        lax.fori_loop(0, TPW, body, 0)
        pltpu.sync_copy(rows, out1d.at[pl.ds(base * D, TPW * D)])

    return gather_k


def _ln_body(emb_ref, pe_ref, scale_ref, bias_ref, out_ref):
    e = emb_ref[...] + pe_ref[...]
    mu = jnp.mean(e, axis=1, keepdims=True)
    d = e - mu
    var = jnp.mean(d * d, axis=1, keepdims=True)
    inv = lax.rsqrt(var + 1e-10)
    out_ref[...] = d * inv * scale_ref[...] + bias_ref[...]


def _ln(emb, pe, scale, bias):
    blk = 1024
    return pl.pallas_call(
        _ln_body,
        grid=(N // blk,),
        in_specs=[
            pl.BlockSpec((blk, D), lambda i: (i, 0)),
            pl.BlockSpec((blk, D), lambda i: (i, 0)),
            pl.BlockSpec((1, D), lambda i: (0, 0)),
            pl.BlockSpec((1, D), lambda i: (0, 0)),
        ],
        out_specs=pl.BlockSpec((blk, D), lambda i: (i, 0)),
        out_shape=jax.ShapeDtypeStruct((N, D), jnp.float32),
    )(emb, pe, scale, bias)


def kernel(x, table, ln_scale, ln_bias):
    idx = x.reshape(-1).astype(jnp.int32)
    q = idx // LANES
    m = idx % LANES
    table_t = table.T
    emb1d = _make_gather()(table_t, q, m)
    emb = emb1d.reshape(N, D)
    pe = jnp.asarray(_PE_FULL)
    out = _ln(emb, pe, ln_scale.reshape(1, D), ln_bias.reshape(1, D))
    return out.reshape(SEQ, BATCH, D)


# speedup vs baseline: 5.5074x; 1.0946x over previous
import functools

import jax
import jax.numpy as jnp
import numpy as np
from jax import lax
from jax.experimental import pallas as pl
from jax.experimental.pallas import tpu as pltpu
from jax.experimental.pallas import tpu_sc as plsc

VOCAB = 1000000
D = 64
SEQ = 2048
BATCH = 4
N = SEQ * BATCH

NC = 2
NS = 16
NW = NC * NS
TPW = N // NW
SPW = SEQ // (NW // BATCH)
LANES = 128
RING = 6
IPAD = TPW + 16


def _pos_enc_np(max_len, d_model):
    position = np.arange(max_len, dtype=np.float32)[:, None]
    div_term = np.exp(
        np.arange(0, d_model, 2, dtype=np.float32) * -(np.log(10000.0) / d_model))
    pe = np.zeros((max_len, d_model), dtype=np.float32)
    pe[:, 0::2] = np.sin(position * div_term)
    pe[:, 1::2] = np.cos(position * div_term)
    return pe


_PE_T = np.ascontiguousarray(_pos_enc_np(SEQ, D).T)[None]


@functools.cache
def _make_gather():
    mesh = plsc.VectorSubcoreMesh(core_axis_name="c", subcore_axis_name="s")

    @functools.partial(
        pl.kernel,
        mesh=mesh,
        out_type=jax.ShapeDtypeStruct((BATCH, D, SEQ), jnp.float32),
        scratch_types=[
            pltpu.VMEM((IPAD,), jnp.int32),
            pltpu.VMEM((IPAD,), jnp.int32),
            pltpu.VMEM((RING, D, LANES), jnp.float32),
            pltpu.VMEM((D, TPW), jnp.float32),
            pltpu.SemaphoreType.DMA,
        ],
        compiler_params=pltpu.CompilerParams(needs_layout_passes=False),
    )
    def gather_k(tab_t, q_hbm, m_hbm, out3, q_v, m_v, ring, trows, sem):
        wid = lax.axis_index("s") * NC + lax.axis_index("c")
        base = wid * TPW
        pltpu.sync_copy(q_hbm.at[pl.ds(base, TPW)], q_v.at[pl.ds(0, TPW)])
        pltpu.sync_copy(m_hbm.at[pl.ds(base, TPW)], m_v.at[pl.ds(0, TPW)])

        def fetch(t, slot):
            qv = q_v[pl.ds(t, 16)][0]
            off = pl.multiple_of(qv * LANES, LANES)
            pltpu.async_copy(
                tab_t.at[:, pl.ds(off, LANES)], ring.at[slot], sem)

        for r in range(RING):
            fetch(r, r)

        lanes16 = lax.iota(jnp.int32, 16)

        def body(t, carry):
            slot = lax.rem(t, RING)
            pltpu.make_async_copy(
                tab_t.at[:, pl.ds(0, LANES)], ring.at[slot], sem).wait()
            m = m_v[pl.ds(t, 16)][0]
            midx = jnp.full((16,), m, jnp.int32)
            sidx = jnp.full((16,), slot, jnp.int32)
            tidx = jnp.full((16,), t, jnp.int32)
            for k in range(D // 16):
                g = plsc.load_gather(ring, [sidx, lanes16 + 16 * k, midx])
                plsc.store_scatter(trows, [lanes16 + 16 * k, tidx], g)
            tn = t + RING

            @pl.when(tn < TPW)
            def _():
                fetch(tn, lax.rem(tn, RING))

            return carry

        lax.fori_loop(0, TPW, body, 0)
        b = wid // (NW // BATCH)
        s0 = lax.rem(wid, NW // BATCH) * SPW
        pltpu.sync_copy(trows, out3.at[b].at[:, pl.ds(s0, SPW)])

    return gather_k


def _ln_body(emb_ref, pe_ref, scale_ref, bias_ref, out_ref):
    e = emb_ref[...] + pe_ref[...]
    mu = jnp.mean(e, axis=1, keepdims=True)
    d = e - mu
    var = jnp.mean(d * d, axis=1, keepdims=True)
    inv = lax.rsqrt(var + 1e-10)
    out_ref[...] = d * inv * scale_ref[...] + bias_ref[...]


def _ln(emb3, pe3, scale, bias):
    blk = 1024
    return pl.pallas_call(
        _ln_body,
        grid=(BATCH, SEQ // blk),
        in_specs=[
            pl.BlockSpec((1, D, blk), lambda b, j: (b, 0, j)),
            pl.BlockSpec((1, D, blk), lambda b, j: (0, 0, j)),
            pl.BlockSpec((1, D, 1), lambda b, j: (0, 0, 0)),
            pl.BlockSpec((1, D, 1), lambda b, j: (0, 0, 0)),
        ],
        out_specs=pl.BlockSpec((1, D, blk), lambda b, j: (b, 0, j)),
        out_shape=jax.ShapeDtypeStruct((BATCH, D, SEQ), jnp.float32),
    )(emb3, pe3, scale, bias)


def kernel(x, table, ln_scale, ln_bias):
    idx = x.T.reshape(-1).astype(jnp.int32)
    q = idx // LANES
    m = idx % LANES
    table_t = table.T
    emb3 = _make_gather()(table_t, q, m)
    pe3 = jnp.asarray(_PE_T)
    out3 = _ln(emb3, pe3, ln_scale.reshape(1, D, 1), ln_bias.reshape(1, D, 1))
    return out3.transpose(2, 0, 1)
